# baseline (speedup 1.0000x reference)
import functools

import jax
import jax.numpy as jnp
from jax import lax
from jax.experimental import pallas as pl
from jax.experimental.pallas import tpu as pltpu
from jax.experimental.pallas import tpu_sc as plsc

B, C, N, T = 8, 64, 10000, 12
P = N * T
BLK_N = 128
BLK_P = BLK_N * T
NBLK = (N + BLK_N - 1) // BLK_N
TIME_V = 288 + 1
DAY_V = 7 + 1

NW = 32
CHUNK = 128
CHW = 3
RPW = CHUNK * CHW
R = NW * RPW
CW = 128


def _sc_gather_body(nid_ref, deg_ref, ntab_ref, dtab_ref,
                    nrows_ref, drows_ref,
                    nid_v, deg_v, nbuf, dbuf, sem):
    wid = lax.axis_index("s") * 2 + lax.axis_index("c")
    base = wid * RPW
    pltpu.sync_copy(nid_ref.at[wid], nid_v)
    pltpu.sync_copy(deg_ref.at[wid], deg_v)
    copies = []
    for j in range(CHW):
        dst = nbuf.at[pl.ds(j * CHUNK, CHUNK)]
        copies.append(pltpu.async_copy(ntab_ref.at[nid_v.at[j]], dst, sem))
        dst = dbuf.at[pl.ds(j * CHUNK, CHUNK)]
        copies.append(pltpu.async_copy(dtab_ref.at[deg_v.at[j]], dst, sem))
    for cp in copies:
        cp.wait()
    pltpu.sync_copy(nbuf, nrows_ref.at[pl.ds(base, RPW)])
    pltpu.sync_copy(dbuf, drows_ref.at[pl.ds(base, RPW)])


@functools.cache
def _make_sc_gather():
    return pl.kernel(
        _sc_gather_body,
        out_type=[jax.ShapeDtypeStruct((R, CW), jnp.float32),
                  jax.ShapeDtypeStruct((R, CW), jnp.float32)],
        mesh=plsc.VectorSubcoreMesh(core_axis_name="c", subcore_axis_name="s"),
        scratch_types=[pltpu.VMEM((CHW, CHUNK), jnp.int32),
                       pltpu.VMEM((CHW, CHUNK), jnp.int32),
                       pltpu.VMEM((RPW, CW), jnp.float32),
                       pltpu.VMEM((RPW, CW), jnp.float32),
                       pltpu.SemaphoreType.DMA],
    )


def _bt_prep_body(idx_ref, tt_ref, dt_ref, out_ref):
    ids = idx_ref[0]
    t_ids = ids[0:1, :]
    d_ids = ids[1:2, :]
    t_iota = lax.broadcasted_iota(jnp.int32, (TIME_V, BLK_P), 0)
    d_iota = lax.broadcasted_iota(jnp.int32, (DAY_V, BLK_P), 0)
    t_oh = (t_iota == t_ids).astype(jnp.float32)
    d_oh = (d_iota == d_ids).astype(jnp.float32)
    bt = lax.dot_general(tt_ref[...], t_oh, (((0,), (0,)), ((), ())),
                         preferred_element_type=jnp.float32)
    bt = bt + lax.dot_general(dt_ref[...], d_oh, (((0,), (0,)), ((), ())),
                              preferred_element_type=jnp.float32)
    out_ref[0] = bt


def _bt_prep(idx_exp, time_table, day_table):
    return pl.pallas_call(
        _bt_prep_body,
        grid=(B,),
        in_specs=[
            pl.BlockSpec((1, 2, BLK_P), lambda b: (b, 0, 0)),
            pl.BlockSpec((TIME_V, C), lambda b: (0, 0)),
            pl.BlockSpec((DAY_V, C), lambda b: (0, 0)),
        ],
        out_specs=pl.BlockSpec((1, C, BLK_P), lambda b: (b, 0, 0)),
        out_shape=jax.ShapeDtypeStruct((B, C, BLK_P), jnp.float32),
    )(idx_exp, time_table, day_table)


def _main_body(x_ref, bt_ref, nr_ref, dr_ref, e_ref, out_ref):
    nadd = nr_ref[:, :C] + dr_ref[:, :C]
    nexp = lax.dot_general(nadd, e_ref[...], (((0,), (0,)), ((), ())),
                           preferred_element_type=jnp.float32)
    out_ref[...] = x_ref[...] + bt_ref[...] + nexp[None, :, :]


def _main(x3, bt, node_rows, deg_rows, expand):
    return pl.pallas_call(
        _main_body,
        grid=(NBLK,),
        in_specs=[
            pl.BlockSpec((B, C, BLK_P), lambda j: (0, 0, j)),
            pl.BlockSpec((B, C, BLK_P), lambda j: (0, 0, 0)),
            pl.BlockSpec((BLK_N, CW), lambda j: (j, 0)),
            pl.BlockSpec((BLK_N, CW), lambda j: (j, 0)),
            pl.BlockSpec((BLK_N, BLK_P), lambda j: (0, 0)),
        ],
        out_specs=pl.BlockSpec((B, C, BLK_P), lambda j: (0, 0, j)),
        out_shape=jax.ShapeDtypeStruct((B, C, P), jnp.float32),
    )(x3, bt, node_rows, deg_rows, expand)


def kernel(x, idx, node_ids, degrees, time_table, day_table, node_table,
           degree_table):
    x3 = x.reshape(B, C, P)
    pad = R - N
    nid = jnp.concatenate(
        [node_ids, jnp.zeros((pad,), jnp.int32)]).reshape(NW, CHW, CHUNK)
    deg = jnp.concatenate(
        [degrees, jnp.zeros((pad,), jnp.int32)]).reshape(NW, CHW, CHUNK)
    ntab_p = jnp.pad(node_table, ((0, 0), (0, CW - C)))
    dtab_p = jnp.pad(degree_table, ((0, 0), (0, CW - C)))
    node_rows, deg_rows = _make_sc_gather()(nid, deg, ntab_p, dtab_p)
    idx_exp = jnp.tile(idx, (1, 1, BLK_N))
    bt = _bt_prep(idx_exp, time_table, day_table)
    expand = (jnp.arange(BLK_P, dtype=jnp.int32)[None, :] // T
              == jnp.arange(BLK_N, dtype=jnp.int32)[:, None]
              ).astype(jnp.float32)
    out3 = _main(x3, bt, node_rows, deg_rows, expand)
    return out3.reshape(B, C, N, T)


# speedup vs baseline: 3.8416x; 3.8416x over previous
import functools

import jax
import jax.numpy as jnp
from jax import lax
from jax.experimental import pallas as pl
from jax.experimental.pallas import tpu as pltpu
from jax.experimental.pallas import tpu_sc as plsc

B, C, N, T = 8, 64, 10000, 12
BT = B * T
TIME_V = 288 + 1
DAY_V = 7 + 1

NW = 32
CHUNK = 128
CHW = 3
RPW = CHUNK * CHW
R = NW * RPW
CW = 128
NBLK = (N + CHUNK - 1) // CHUNK
NPAD = NBLK * CHUNK


def _sc_gather_body(nid_ref, deg_ref, ntab_ref, dtab_ref,
                    nrows_ref, drows_ref,
                    nid_v, deg_v, nbuf, dbuf, sem):
    wid = lax.axis_index("s") * 2 + lax.axis_index("c")
    base = wid * RPW
    pltpu.sync_copy(nid_ref.at[wid], nid_v)
    pltpu.sync_copy(deg_ref.at[wid], deg_v)
    copies = []
    for j in range(CHW):
        dst = nbuf.at[pl.ds(j * CHUNK, CHUNK)]
        copies.append(pltpu.async_copy(ntab_ref.at[nid_v.at[j]], dst, sem))
        dst = dbuf.at[pl.ds(j * CHUNK, CHUNK)]
        copies.append(pltpu.async_copy(dtab_ref.at[deg_v.at[j]], dst, sem))
    for cp in copies:
        cp.wait()
    pltpu.sync_copy(nbuf, nrows_ref.at[pl.ds(base, RPW)])
    pltpu.sync_copy(dbuf, drows_ref.at[pl.ds(base, RPW)])


@functools.cache
def _make_sc_gather():
    return pl.kernel(
        _sc_gather_body,
        out_type=[jax.ShapeDtypeStruct((R, CW), jnp.float32),
                  jax.ShapeDtypeStruct((R, CW), jnp.float32)],
        mesh=plsc.VectorSubcoreMesh(core_axis_name="c", subcore_axis_name="s"),
        scratch_types=[pltpu.VMEM((CHW, CHUNK), jnp.int32),
                       pltpu.VMEM((CHW, CHUNK), jnp.int32),
                       pltpu.VMEM((RPW, CW), jnp.float32),
                       pltpu.VMEM((RPW, CW), jnp.float32),
                       pltpu.SemaphoreType.DMA],
    )


def _bt_prep_body(iflat_ref, tt_ref, dt_ref, out_ref):
    ids = iflat_ref[...]
    t_ids = ids[0:1, :]
    d_ids = ids[1:2, :]
    t_iota = lax.broadcasted_iota(jnp.int32, (TIME_V, BT), 0)
    d_iota = lax.broadcasted_iota(jnp.int32, (DAY_V, BT), 0)
    t_oh = (t_iota == t_ids).astype(jnp.float32)
    d_oh = (d_iota == d_ids).astype(jnp.float32)
    bt = lax.dot_general(t_oh, tt_ref[...], (((0,), (0,)), ((), ())),
                         preferred_element_type=jnp.float32)
    bt = bt + lax.dot_general(d_oh, dt_ref[...], (((0,), (0,)), ((), ())),
                              preferred_element_type=jnp.float32)
    out_ref[...] = bt[:, :, None]


def _bt_prep(iflat, time_table, day_table):
    return pl.pallas_call(
        _bt_prep_body,
        grid=(1,),
        in_specs=[
            pl.BlockSpec((2, BT), lambda i: (0, 0)),
            pl.BlockSpec((TIME_V, C), lambda i: (0, 0)),
            pl.BlockSpec((DAY_V, C), lambda i: (0, 0)),
        ],
        out_specs=pl.BlockSpec((BT, C, 1), lambda i: (0, 0, 0)),
        out_shape=jax.ShapeDtypeStruct((BT, C, 1), jnp.float32),
    )(iflat, time_table, day_table)


def _tsum_body(nr_ref, dr_ref, eye_ref, out_ref):
    s = nr_ref[:, :C] + dr_ref[:, :C]
    out_ref[...] = lax.dot_general(s, eye_ref[...], (((0,), (0,)), ((), ())),
                                   preferred_element_type=jnp.float32)


def _tsum(node_rows, deg_rows, eye):
    return pl.pallas_call(
        _tsum_body,
        grid=(NBLK,),
        in_specs=[
            pl.BlockSpec((CHUNK, CW), lambda i: (i, 0)),
            pl.BlockSpec((CHUNK, CW), lambda i: (i, 0)),
            pl.BlockSpec((CHUNK, CHUNK), lambda i: (0, 0)),
        ],
        out_specs=pl.BlockSpec((C, CHUNK), lambda i: (0, i)),
        out_shape=jax.ShapeDtypeStruct((C, N), jnp.float32),
    )(node_rows, deg_rows, eye)


def _main_body(x_ref, bt_ref, nadd_ref, out_ref):
    out_ref[...] = x_ref[...] + bt_ref[...] + nadd_ref[...][None, :, :]


def _main(y, btcol, nadd_t):
    return pl.pallas_call(
        _main_body,
        grid=(BT,),
        in_specs=[
            pl.BlockSpec((1, C, N), lambda i: (i, 0, 0)),
            pl.BlockSpec((1, C, 1), lambda i: (i, 0, 0)),
            pl.BlockSpec((C, N), lambda i: (0, 0)),
        ],
        out_specs=pl.BlockSpec((1, C, N), lambda i: (i, 0, 0)),
        out_shape=jax.ShapeDtypeStruct((BT, C, N), jnp.float32),
    )(y, btcol, nadd_t)


def kernel(x, idx, node_ids, degrees, time_table, day_table, node_table,
           degree_table):
    y = jnp.transpose(x, (0, 3, 1, 2)).reshape(BT, C, N)
    pad = R - N
    nid = jnp.concatenate(
        [node_ids, jnp.zeros((pad,), jnp.int32)]).reshape(NW, CHW, CHUNK)
    deg = jnp.concatenate(
        [degrees, jnp.zeros((pad,), jnp.int32)]).reshape(NW, CHW, CHUNK)
    ntab_p = jnp.pad(node_table, ((0, 0), (0, CW - C)))
    dtab_p = jnp.pad(degree_table, ((0, 0), (0, CW - C)))
    node_rows, deg_rows = _make_sc_gather()(nid, deg, ntab_p, dtab_p)
    iflat = jnp.stack([idx[:, 0, :].reshape(BT), idx[:, 1, :].reshape(BT)])
    btcol = _bt_prep(iflat, time_table, day_table)
    eye = jnp.eye(CHUNK, dtype=jnp.float32)
    nadd_t = _tsum(node_rows, deg_rows, eye)
    out = _main(y, btcol, nadd_t)
    return jnp.transpose(out.reshape(B, T, C, N), (0, 2, 3, 1))


# speedup vs baseline: 5.3392x; 1.3898x over previous
import functools

import jax
import jax.numpy as jnp
from jax import lax
from jax.experimental import pallas as pl
from jax.experimental.pallas import tpu as pltpu
from jax.experimental.pallas import tpu_sc as plsc

B, C, N, T = 8, 64, 10000, 12
BT = B * T
TIME_V = 288 + 1
DAY_V = 7 + 1
DEG_V = 64 + 1

NW = 32
RPW = 320
R = NW * RPW
CW = 128
CHUNK = 128
NBLK = (N + CHUNK - 1) // CHUNK


def _sc_gather_body(nid_ref, ntab_ref, nrows_ref, nidx, nbuf, sem):
    wid = lax.axis_index("s") * 2 + lax.axis_index("c")
    base = wid * RPW
    pltpu.sync_copy(nid_ref.at[wid], nidx)
    copies = []
    for lo in range(0, RPW, CHUNK):
        sz = min(CHUNK, RPW - lo)
        copies.append(pltpu.async_copy(
            ntab_ref.at[nidx.at[pl.ds(lo, sz)]],
            nbuf.at[pl.ds(lo, sz)], sem))
    for cp in copies:
        cp.wait()
    pltpu.sync_copy(nbuf, nrows_ref.at[pl.ds(base, RPW)])


@functools.cache
def _make_sc_gather():
    return pl.kernel(
        _sc_gather_body,
        out_type=jax.ShapeDtypeStruct((R, CW), jnp.float32),
        mesh=plsc.VectorSubcoreMesh(core_axis_name="c", subcore_axis_name="s"),
        scratch_types=[pltpu.VMEM((RPW,), jnp.int32),
                       pltpu.VMEM((RPW, CW), jnp.float32),
                       pltpu.SemaphoreType.DMA],
    )


def _bt_prep_body(iflat_ref, tt_ref, dt_ref, out_ref):
    ids = iflat_ref[...]
    t_ids = ids[0:1, :]
    d_ids = ids[1:2, :]
    t_iota = lax.broadcasted_iota(jnp.int32, (TIME_V, BT), 0)
    d_iota = lax.broadcasted_iota(jnp.int32, (DAY_V, BT), 0)
    t_oh = (t_iota == t_ids).astype(jnp.float32)
    d_oh = (d_iota == d_ids).astype(jnp.float32)
    bt = lax.dot_general(t_oh, tt_ref[...], (((0,), (0,)), ((), ())),
                         preferred_element_type=jnp.float32)
    bt = bt + lax.dot_general(d_oh, dt_ref[...], (((0,), (0,)), ((), ())),
                              preferred_element_type=jnp.float32)
    out_ref[...] = bt[:, :, None]


def _bt_prep(iflat, time_table, day_table):
    return pl.pallas_call(
        _bt_prep_body,
        grid=(1,),
        in_specs=[
            pl.BlockSpec((2, BT), lambda i: (0, 0)),
            pl.BlockSpec((TIME_V, C), lambda i: (0, 0)),
            pl.BlockSpec((DAY_V, C), lambda i: (0, 0)),
        ],
        out_specs=pl.BlockSpec((BT, C, 1), lambda i: (0, 0, 0)),
        out_shape=jax.ShapeDtypeStruct((BT, C, 1), jnp.float32),
    )(iflat, time_table, day_table)


def _tsum_body(nr_ref, deg_ref, dtab_ref, eye_ref, out_ref):
    s = nr_ref[:, :C]
    nt = lax.dot_general(s, eye_ref[...], (((0,), (0,)), ((), ())),
                         preferred_element_type=jnp.float32)
    d_ids = deg_ref[0]
    d_iota = lax.broadcasted_iota(jnp.int32, (DEG_V, CHUNK), 0)
    d_oh = (d_iota == d_ids).astype(jnp.float32)
    dt = lax.dot_general(dtab_ref[...], d_oh, (((0,), (0,)), ((), ())),
                         preferred_element_type=jnp.float32)
    out_ref[...] = nt + dt


def _tsum(node_rows, deg3, dtab, eye):
    return pl.pallas_call(
        _tsum_body,
        grid=(NBLK,),
        in_specs=[
            pl.BlockSpec((CHUNK, CW), lambda i: (i, 0)),
            pl.BlockSpec((1, 1, CHUNK), lambda i: (i, 0, 0)),
            pl.BlockSpec((DEG_V, C), lambda i: (0, 0)),
            pl.BlockSpec((CHUNK, CHUNK), lambda i: (0, 0)),
        ],
        out_specs=pl.BlockSpec((C, CHUNK), lambda i: (0, i)),
        out_shape=jax.ShapeDtypeStruct((C, N), jnp.float32),
    )(node_rows, deg3, dtab, eye)


def _main_body(x_ref, bt_ref, nadd_ref, out_ref):
    out_ref[...] = x_ref[...] + bt_ref[...] + nadd_ref[...][None, :, :]


def _main(y, btcol, nadd_t):
    return pl.pallas_call(
        _main_body,
        grid=(BT,),
        in_specs=[
            pl.BlockSpec((1, C, N), lambda i: (i, 0, 0)),
            pl.BlockSpec((1, C, 1), lambda i: (i, 0, 0)),
            pl.BlockSpec((C, N), lambda i: (0, 0)),
        ],
        out_specs=pl.BlockSpec((1, C, N), lambda i: (i, 0, 0)),
        out_shape=jax.ShapeDtypeStruct((BT, C, N), jnp.float32),
    )(y, btcol, nadd_t)


def kernel(x, idx, node_ids, degrees, time_table, day_table, node_table,
           degree_table):
    y = jnp.transpose(x, (0, 3, 1, 2)).reshape(BT, C, N)
    nid = jnp.concatenate(
        [node_ids, jnp.zeros((R - N,), jnp.int32)]).reshape(NW, RPW)
    ntab_p = jnp.pad(node_table, ((0, 0), (0, CW - C)))
    node_rows = _make_sc_gather()(nid, ntab_p)
    iflat = jnp.stack([idx[:, 0, :].reshape(BT), idx[:, 1, :].reshape(BT)])
    btcol = _bt_prep(iflat, time_table, day_table)
    deg3 = jnp.concatenate(
        [degrees, jnp.zeros((NBLK * CHUNK - N,), jnp.int32)]
    ).reshape(NBLK, 1, CHUNK)
    eye = jnp.eye(CHUNK, dtype=jnp.float32)
    nadd_t = _tsum(node_rows, deg3, degree_table, eye)
    out = _main(y, btcol, nadd_t)
    return jnp.transpose(out.reshape(B, T, C, N), (0, 2, 3, 1))


# speedup vs baseline: 5.6406x; 1.0564x over previous
import functools

import jax
import jax.numpy as jnp
from jax import lax
from jax.experimental import pallas as pl
from jax.experimental.pallas import tpu as pltpu
from jax.experimental.pallas import tpu_sc as plsc

B, C, N, T = 8, 64, 10000, 12
BT = B * T
TIME_V = 288 + 1
DAY_V = 7 + 1
DEG_V = 64 + 1

NW = 32
RPW = 320
R = NW * RPW
CW = 128
CHUNK = 128
NBLK = (N + CHUNK - 1) // CHUNK


def _sc_gather_body(nid_ref, ntab_ref, nrows_ref, nidx, nbuf, sem):
    wid = lax.axis_index("s") * 2 + lax.axis_index("c")
    base = wid * RPW
    pltpu.sync_copy(nid_ref.at[wid], nidx)
    copies = []
    for lo in range(0, RPW, CHUNK):
        sz = min(CHUNK, RPW - lo)
        copies.append(pltpu.async_copy(
            ntab_ref.at[nidx.at[pl.ds(lo, sz)]],
            nbuf.at[pl.ds(lo, sz)], sem))
    for cp in copies:
        cp.wait()
    pltpu.sync_copy(nbuf, nrows_ref.at[pl.ds(base, RPW)])


@functools.cache
def _make_sc_gather():
    return pl.kernel(
        _sc_gather_body,
        out_type=jax.ShapeDtypeStruct((R, CW), jnp.float32),
        mesh=plsc.VectorSubcoreMesh(core_axis_name="c", subcore_axis_name="s"),
        scratch_types=[pltpu.VMEM((RPW,), jnp.int32),
                       pltpu.VMEM((RPW, CW), jnp.float32),
                       pltpu.SemaphoreType.DMA],
    )


def _bt_prep_body(iflat_ref, tt_ref, dt_ref, out_ref):
    ids = iflat_ref[...]
    t_ids = ids[0:1, :]
    d_ids = ids[1:2, :]
    t_iota = lax.broadcasted_iota(jnp.int32, (TIME_V, BT), 0)
    d_iota = lax.broadcasted_iota(jnp.int32, (DAY_V, BT), 0)
    t_oh = (t_iota == t_ids).astype(jnp.float32)
    d_oh = (d_iota == d_ids).astype(jnp.float32)
    bt = lax.dot_general(t_oh, tt_ref[...], (((0,), (0,)), ((), ())),
                         preferred_element_type=jnp.float32)
    bt = bt + lax.dot_general(d_oh, dt_ref[...], (((0,), (0,)), ((), ())),
                              preferred_element_type=jnp.float32)
    out_ref[...] = bt[:, :, None]


def _bt_prep(iflat, time_table, day_table):
    return pl.pallas_call(
        _bt_prep_body,
        grid=(1,),
        in_specs=[
            pl.BlockSpec((2, BT), lambda i: (0, 0)),
            pl.BlockSpec((TIME_V, C), lambda i: (0, 0)),
            pl.BlockSpec((DAY_V, C), lambda i: (0, 0)),
        ],
        out_specs=pl.BlockSpec((BT, C, 1), lambda i: (0, 0, 0)),
        out_shape=jax.ShapeDtypeStruct((BT, C, 1), jnp.float32),
    )(iflat, time_table, day_table)


def _tsum_body(nr_ref, deg_ref, dtab_ref, eye_ref, out_ref):
    s = nr_ref[:, :C]
    nt = lax.dot_general(s, eye_ref[...], (((0,), (0,)), ((), ())),
                         preferred_element_type=jnp.float32)
    d_ids = deg_ref[0]
    d_iota = lax.broadcasted_iota(jnp.int32, (DEG_V, CHUNK), 0)
    d_oh = (d_iota == d_ids).astype(jnp.float32)
    dt = lax.dot_general(dtab_ref[...], d_oh, (((0,), (0,)), ((), ())),
                         preferred_element_type=jnp.float32)
    out_ref[...] = nt + dt


def _tsum(node_rows, deg3, dtab, eye):
    return pl.pallas_call(
        _tsum_body,
        grid=(NBLK,),
        in_specs=[
            pl.BlockSpec((CHUNK, CW), lambda i: (i, 0)),
            pl.BlockSpec((1, 1, CHUNK), lambda i: (i, 0, 0)),
            pl.BlockSpec((DEG_V, C), lambda i: (0, 0)),
            pl.BlockSpec((CHUNK, CHUNK), lambda i: (0, 0)),
        ],
        out_specs=pl.BlockSpec((C, CHUNK), lambda i: (0, i)),
        out_shape=jax.ShapeDtypeStruct((C, N), jnp.float32),
    )(node_rows, deg3, dtab, eye)


RB = 4


def _main_body(x_ref, bt_ref, nadd_ref, out_ref):
    out_ref[...] = x_ref[...] + bt_ref[...] + nadd_ref[...][None, :, :]


def _main(y, btcol, nadd_t):
    return pl.pallas_call(
        _main_body,
        grid=(BT // RB,),
        in_specs=[
            pl.BlockSpec((RB, C, N), lambda i: (i, 0, 0)),
            pl.BlockSpec((RB, C, 1), lambda i: (i, 0, 0)),
            pl.BlockSpec((C, N), lambda i: (0, 0)),
        ],
        out_specs=pl.BlockSpec((RB, C, N), lambda i: (i, 0, 0)),
        out_shape=jax.ShapeDtypeStruct((BT, C, N), jnp.float32),
    )(y, btcol, nadd_t)


def kernel(x, idx, node_ids, degrees, time_table, day_table, node_table,
           degree_table):
    y = jnp.transpose(x, (0, 3, 1, 2)).reshape(BT, C, N)
    nid = jnp.concatenate(
        [node_ids, jnp.zeros((R - N,), jnp.int32)]).reshape(NW, RPW)
    ntab_p = jnp.pad(node_table, ((0, 0), (0, CW - C)))
    node_rows = _make_sc_gather()(nid, ntab_p)
    iflat = jnp.stack([idx[:, 0, :].reshape(BT), idx[:, 1, :].reshape(BT)])
    btcol = _bt_prep(iflat, time_table, day_table)
    deg3 = jnp.concatenate(
        [degrees, jnp.zeros((NBLK * CHUNK - N,), jnp.int32)]
    ).reshape(NBLK, 1, CHUNK)
    eye = jnp.eye(CHUNK, dtype=jnp.float32)
    nadd_t = _tsum(node_rows, deg3, degree_table, eye)
    out = _main(y, btcol, nadd_t)
    return jnp.transpose(out.reshape(B, T, C, N), (0, 2, 3, 1))
